# Initial kernel scaffold; baseline (speedup 1.0000x reference)
import jax
import jax.numpy as jnp
from jax.experimental import pallas as pl


def kernel(x_num, x_cat, tables, W1, b1, W2, b2, W3, b3):
    raise NotImplementedError("write your pallas kernel here")


# speedup vs baseline: 2.0038x; 2.0038x over previous
import jax
import jax.numpy as jnp
from jax import lax
from jax.experimental import pallas as pl
from jax.experimental.pallas import tpu as pltpu
from jax.experimental.pallas import tpu_sc as plsc

_N_FIELDS = 26
_VOCAB = 100000
_EMB = 32
_N_NUM = 13
_B = 16384
_ROWS = _B * _N_FIELDS
_NC, _NS = 2, 16
_NW = _NC * _NS
_RPW = _ROWS // _NW
_IDXW = 128
_CH = 1664
_NCHUNK = _RPW // _CH
_GPC = _CH // _IDXW
_IPW = _RPW // _IDXW


def _gather_body(tab_ref, idx_ref, out_ref, idx_v, rows_v, sem0, sem1):
    wid = lax.axis_index("c") * _NS + lax.axis_index("s")
    base = wid * _RPW
    pltpu.sync_copy(idx_ref.at[pl.ds(wid * _IPW, _IPW)], idx_v)
    sems = (sem0, sem1)

    def fire(c):
        p = c % 2
        return [
            pltpu.async_copy(
                tab_ref.at[idx_v.at[c * _GPC + j]],
                rows_v.at[p, pl.ds(j * _IDXW, _IDXW)],
                sems[p])
            for j in range(_GPC)
        ]

    pending = {0: fire(0)}
    for c in range(_NCHUNK):
        if c + 1 < _NCHUNK:
            pending[c + 1] = fire(c + 1)
        for d in pending.pop(c):
            d.wait()
        pltpu.sync_copy(rows_v.at[c % 2],
                        out_ref.at[pl.ds(base + c * _CH, _CH)])


def _sc_gather(tables_flat, idx2d):
    mesh = plsc.VectorSubcoreMesh(core_axis_name="c", subcore_axis_name="s")
    k = pl.kernel(
        _gather_body,
        out_type=jax.ShapeDtypeStruct((_ROWS, _EMB), jnp.float32),
        mesh=mesh,
        scratch_types=[
            pltpu.VMEM((_IPW, _IDXW), jnp.int32),
            pltpu.VMEM((2, _CH, _EMB), jnp.float32),
            pltpu.SemaphoreType.DMA,
            pltpu.SemaphoreType.DMA,
        ],
        compiler_params=pltpu.CompilerParams(use_tc_tiling_on_sc=False),
    )
    return k(tables_flat, idx2d)


_BLK = 2048


def _mlp_body(xn_ref, emb_ref, w1n_ref, w1e_ref, b1_ref, w2_ref, b2_ref,
              w3_ref, b3_ref, out_ref):
    h = jnp.dot(emb_ref[...], w1e_ref[...], preferred_element_type=jnp.float32)
    h = h + jnp.dot(xn_ref[...], w1n_ref[...],
                    preferred_element_type=jnp.float32)
    h = jnp.maximum(h + b1_ref[...], 0.0)
    h = jnp.maximum(
        jnp.dot(h, w2_ref[...], preferred_element_type=jnp.float32)
        + b2_ref[...], 0.0)
    out_ref[...] = (jnp.dot(h, w3_ref[...], preferred_element_type=jnp.float32)
                    + b3_ref[...])


def _mlp(x_num, emb, W1n, W1e, b1, W2, b2, W3, b3):
    grid = (_B // _BLK,)
    return pl.pallas_call(
        _mlp_body,
        grid=grid,
        in_specs=[
            pl.BlockSpec((_BLK, _N_NUM), lambda i: (i, 0)),
            pl.BlockSpec((_BLK, _N_FIELDS * _EMB), lambda i: (i, 0)),
            pl.BlockSpec((_N_NUM, 128), lambda i: (0, 0)),
            pl.BlockSpec((_N_FIELDS * _EMB, 128), lambda i: (0, 0)),
            pl.BlockSpec((1, 128), lambda i: (0, 0)),
            pl.BlockSpec((128, 64), lambda i: (0, 0)),
            pl.BlockSpec((1, 64), lambda i: (0, 0)),
            pl.BlockSpec((64, 1), lambda i: (0, 0)),
            pl.BlockSpec((1, 1), lambda i: (0, 0)),
        ],
        out_specs=pl.BlockSpec((_BLK, 1), lambda i: (i, 0)),
        out_shape=jax.ShapeDtypeStruct((_B, 1), jnp.float32),
        compiler_params=pltpu.CompilerParams(
            dimension_semantics=("parallel",)),
    )(x_num, emb, W1n, W1e, b1, W2, b2, W3, b3)


def kernel(x_num, x_cat, tables, W1, b1, W2, b2, W3, b3):
    offs = (jnp.arange(_N_FIELDS, dtype=jnp.int32) * _VOCAB)[None, :]
    idx2d = (x_cat.astype(jnp.int32) + offs).reshape(_ROWS // _IDXW, _IDXW)
    emb_flat = _sc_gather(tables.reshape(_N_FIELDS * _VOCAB, _EMB), idx2d)
    emb = emb_flat.reshape(_B, _N_FIELDS * _EMB)
    return _mlp(x_num, emb,
                W1[:_N_NUM], W1[_N_NUM:], b1.reshape(1, 128),
                W2, b2.reshape(1, 64), W3, b3.reshape(1, 1))
